# Initial kernel scaffold; baseline (speedup 1.0000x reference)
import jax
import jax.numpy as jnp
from jax.experimental import pallas as pl


def kernel(X, tables, adj):
    raise NotImplementedError("write your pallas kernel here")


# speedup vs baseline: 1.1044x; 1.1044x over previous
import jax
import jax.numpy as jnp
from jax import lax
from jax.experimental import pallas as pl
from jax.experimental.pallas import tpu as pltpu
from jax.experimental.pallas import tpu_sc as plsc

B, F, V, D, K = 4096, 26, 100000, 32, 10
NC, NS = 2, 16
NW = NC * NS
CB = 32
NCH = B // (CB * NW)
NE = CB * K
VB = V // 4
AR = F * V * K // 128


def _perm(v, idx):
    dn = lax.GatherDimensionNumbers(offset_dims=(), collapsed_slice_dims=(0,),
                                    start_index_map=(0,))
    return lax.gather(v, idx[:, None], dn, slice_sizes=(1,),
                      mode=lax.GatherScatterMode.PROMISE_IN_BOUNDS)


def _recip(v):
    r = lax.bitcast_convert_type(
        jnp.int32(0x7EF311C3) - lax.bitcast_convert_type(v, jnp.int32),
        jnp.float32)
    for _ in range(4):
        r = r * (2.0 - v * r)
    return r


def _body(xblk_hbm, xoff_hbm, c0_hbm, arowi_hbm,
          tab_hbm, adj_hbm, out_hbm,
          xblk_v, xoff_v, c0_v, arow_v,
          adjbuf_v, nblk_v, noff_v, bebuf_v, nbuf_v, out_v,
          sem1, sem2, sem3):
    wid = lax.axis_index("s") * NC + lax.axis_index("c")

    def run_chunk(b0):
        def task(f, _):
            pltpu.sync_copy(xblk_hbm.at[f, pl.ds(b0, CB)], xblk_v)
            pltpu.sync_copy(xoff_hbm.at[f, pl.ds(b0, CB)],
                            xoff_v.at[pl.ds(0, CB)])
            pltpu.sync_copy(c0_hbm.at[f, pl.ds(b0, CB)],
                            c0_v.at[pl.ds(0, CB)])
            pltpu.sync_copy(arowi_hbm.at[f, pl.ds(2 * b0, 2 * CB)], arow_v)
            cp_a = pltpu.async_copy(adj_hbm.at[arow_v], adjbuf_v, sem1)
            cp_b = pltpu.async_copy(tab_hbm.at[f].at[xblk_v], bebuf_v, sem2)
            cp_a.wait()

            def extract(j, _2):
                off = c0_v[pl.ds(j, 16)][0]
                edge = adjbuf_v[j + j, pl.ds(off, 16)]
                no = edge & 3
                nb = ((edge - no).astype(jnp.float32) * 0.25).astype(jnp.int32)
                nblk_v[pl.ds(j * K, 16)] = nb
                noff_v[pl.ds(j * K, 16)] = (
                    no.astype(jnp.float32) * float(D)).astype(jnp.int32)
                return 0

            lax.fori_loop(0, CB, extract, 0)
            cps = [pltpu.async_copy(
                       tab_hbm.at[f].at[nblk_v.at[pl.ds(c * 128, n)]],
                       nbuf_v.at[pl.ds(c * 128, n), :], sem3)
                   for c, n in ((0, 128), (1, 128), (2, 64))]
            cp_b.wait()
            for cp in cps:
                cp.wait()

            lanes = lax.iota(jnp.int32, 16)

            def pair(j, _2):
                ob = xoff_v[pl.ds(j, 16)][0]
                be_lo = bebuf_v[j, pl.ds(ob, 16)]
                be_hi = bebuf_v[j, pl.ds(ob + 16, 16)]
                e0 = j * K
                nv = noff_v[pl.ds(e0, 16)]
                nlo, nhi, dots = [], [], []
                for k in range(K):
                    nb = nv[k]
                    lo = nbuf_v[e0 + k, pl.ds(nb, 16)]
                    hi = nbuf_v[e0 + k, pl.ds(nb + 16, 16)]
                    nlo.append(lo)
                    nhi.append(hi)
                    p = lo * be_lo + hi * be_hi
                    for d in (8, 4, 2, 1):
                        p = p + _perm(p, lanes ^ d)
                    dots.append(p)
                m = dots[0]
                for k in range(1, K):
                    m = jnp.maximum(m, dots[k])
                es = [jnp.exp(dots[k] - m) for k in range(K)]
                den = es[0]
                for k in range(1, K):
                    den = den + es[k]
                inv = _recip(den)
                acc_lo = be_lo
                acc_hi = be_hi
                for k in range(K):
                    w = es[k] * inv
                    acc_lo = acc_lo + w * nlo[k]
                    acc_hi = acc_hi + w * nhi[k]
                out_v[j, pl.ds(f * D, 16)] = acc_lo
                out_v[j, pl.ds(f * D + 16, 16)] = acc_hi
                return 0

            lax.fori_loop(0, CB, pair, 0)
            return 0

        lax.fori_loop(0, F, task, 0)
        pltpu.sync_copy(out_v, out_hbm.at[pl.ds(b0, CB), :])

    for c in range(NCH):
        run_chunk((wid * NCH + c) * CB)


_sc_call = pl.kernel(
    _body,
    out_type=jax.ShapeDtypeStruct((B, F * D), jnp.float32),
    mesh=plsc.VectorSubcoreMesh(core_axis_name="c", subcore_axis_name="s",
                                num_cores=NC, num_subcores=NS),
    scratch_types=[
        pltpu.VMEM((CB,), jnp.int32),
        pltpu.VMEM((CB + 16,), jnp.int32),
        pltpu.VMEM((CB + 16,), jnp.int32),
        pltpu.VMEM((2 * CB,), jnp.int32),
        pltpu.VMEM((2 * CB, 128), jnp.int32),
        pltpu.VMEM((NE + 16,), jnp.int32),
        pltpu.VMEM((NE + 16,), jnp.int32),
        pltpu.VMEM((CB, 128), jnp.float32),
        pltpu.VMEM((NE, 128), jnp.float32),
        pltpu.VMEM((CB, F * D), jnp.float32),
        pltpu.SemaphoreType.DMA,
        pltpu.SemaphoreType.DMA,
        pltpu.SemaphoreType.DMA,
    ],
)


def kernel(X, tables, adj):
    xt = X.T
    foff = jnp.arange(F, dtype=jnp.int32)[:, None] * V
    g = (xt + foff) * K
    r0 = lax.shift_right_logical(g, 7)
    c0 = jnp.bitwise_and(g, 127)
    r1 = jnp.minimum(r0 + 1, AR - 1)
    arowi = jnp.stack([r0, r1], axis=-1).reshape(F, 2 * B)
    xblk = lax.shift_right_logical(xt, 2)
    xoff = jnp.bitwise_and(xt, 3) * D
    tab_g = tables.reshape(F, VB, 128)
    adj_g = adj.reshape(AR, 128)
    out2 = _sc_call(xblk, xoff, c0, arowi, tab_g, adj_g)
    return out2.reshape(B, F, D)
